# baseline (speedup 1.0000x reference)
import functools

import jax
import jax.numpy as jnp
from jax import lax
from jax.experimental import pallas as pl
from jax.experimental.pallas import tpu as pltpu
from jax.experimental.pallas import tpu_sc as plsc

_NUM_CLASSES = 100000
_FEAT_DIM = 64
_BATCH = 16384
_LANES = 16
_NC = 2
_NS = 16
_NW = _NC * _NS
_DPW = _FEAT_DIM // _NW
_FCH = 4096
_NFC = _BATCH // _FCH
_UNROLL = 8

_mesh = plsc.VectorSubcoreMesh(core_axis_name="c", subcore_axis_name="s")


@functools.partial(
    pl.kernel,
    mesh=_mesh,
    out_type=jax.ShapeDtypeStruct((_NW, _LANES), jnp.float32),
    scratch_types=[
        pltpu.VMEM((_BATCH,), jnp.int32),
        pltpu.VMEM((_NUM_CLASSES,), jnp.float32),
        pltpu.VMEM((2, _FCH), jnp.float32),
        pltpu.VMEM((_LANES,), jnp.float32),
        pltpu.VMEM_SHARED((_BATCH,), jnp.int32),
        pltpu.SemaphoreType.DMA,
        pltpu.SemaphoreType.DMA,
        pltpu.SemaphoreType.DMA,
    ],
    compiler_params=pltpu.CompilerParams(needs_layout_passes=False),
)
def _center_loss_sc(label_hbm, feature_t_hbm, centers_t_hbm, out_hbm,
                    lab_v, row_v, fch_v, part_v, lab_sh, rsem, fsem, lsem):
    sid = lax.axis_index("s")
    wid = sid * _NC + lax.axis_index("c")
    d0 = wid * _DPW

    rcopy = pltpu.async_copy(centers_t_hbm.at[d0], row_v, rsem)
    chunks = [(dd, c) for dd in range(_DPW) for c in range(_NFC)]
    fcopies = [
        pltpu.async_copy(
            feature_t_hbm.at[d0 + dd, pl.ds(c * _FCH, _FCH)],
            fch_v.at[k % 2],
            fsem,
        )
        for k, (dd, c) in enumerate(chunks[:2])
    ]
    @pl.when(sid == 0)
    def _():
        pltpu.sync_copy(label_hbm, lab_sh)

    plsc.subcore_barrier()
    lcopy = pltpu.async_copy(lab_sh, lab_v, lsem)

    rcopy.wait()
    lcopy.wait()

    accs = tuple(jnp.zeros((_LANES,), jnp.float32) for _ in range(_UNROLL))
    for k, (dd, c) in enumerate(chunks):
        if c == 0 and dd > 0:
            pltpu.async_copy(centers_t_hbm.at[d0 + dd], row_v, rsem).wait()
        fcopies[k].wait()

        def body(i, a, k=k, c=c):
            new = []
            for u in range(_UNROLL):
                off = i * (_UNROLL * _LANES) + u * _LANES
                lv = lab_v[pl.ds(c * _FCH + off, _LANES)]
                cv = plsc.load_gather(row_v, [lv])
                fv = fch_v[k % 2, pl.ds(off, _LANES)]
                dv = fv - cv
                new.append(a[u] + dv * dv)
            return tuple(new)

        accs = lax.fori_loop(0, _FCH // (_LANES * _UNROLL), body, accs)
        if k + 2 < len(chunks):
            dn, cn = chunks[k + 2]
            fcopies.append(
                pltpu.async_copy(
                    feature_t_hbm.at[d0 + dn, pl.ds(cn * _FCH, _FCH)],
                    fch_v.at[k % 2],
                    fsem,
                )
            )

    tot = accs[0]
    for a in accs[1:]:
        tot = tot + a
    part_v[...] = tot
    pltpu.sync_copy(part_v, out_hbm.at[wid])


def kernel(label, feature, centers):
    lab = label.astype(jnp.int32)
    partials = _center_loss_sc(lab, feature.T, centers.T)
    return jnp.sum(partials) * 0.5


# speedup vs baseline: 1.0122x; 1.0122x over previous
import functools

import jax
import jax.numpy as jnp
from jax import lax
from jax.experimental import pallas as pl
from jax.experimental.pallas import tpu as pltpu
from jax.experimental.pallas import tpu_sc as plsc

_NUM_CLASSES = 100000
_FEAT_DIM = 64
_BATCH = 16384
_LANES = 16
_NC = 2
_NS = 16
_NW = _NC * _NS
_DPW = _FEAT_DIM // _NW
_FCH = 4096
_NFC = _BATCH // _FCH
_UNROLL = 4

_mesh = plsc.VectorSubcoreMesh(core_axis_name="c", subcore_axis_name="s")


@functools.partial(
    pl.kernel,
    mesh=_mesh,
    out_type=jax.ShapeDtypeStruct((_NW, _LANES), jnp.float32),
    scratch_types=[
        pltpu.VMEM((_BATCH,), jnp.int32),
        pltpu.VMEM((_NUM_CLASSES,), jnp.float32),
        pltpu.VMEM((2, _FCH), jnp.float32),
        pltpu.VMEM((_LANES,), jnp.float32),
        pltpu.VMEM_SHARED((_BATCH,), jnp.int32),
        pltpu.SemaphoreType.DMA,
        pltpu.SemaphoreType.DMA,
        pltpu.SemaphoreType.DMA,
    ],
    compiler_params=pltpu.CompilerParams(needs_layout_passes=False),
)
def _center_loss_sc(label_hbm, feature_t_hbm, centers_t_hbm, out_hbm,
                    lab_v, row_v, fch_v, part_v, lab_sh, rsem, fsem, lsem):
    sid = lax.axis_index("s")
    wid = sid * _NC + lax.axis_index("c")
    d0 = wid * _DPW

    rcopy = pltpu.async_copy(centers_t_hbm.at[d0], row_v, rsem)
    chunks = [(dd, c) for dd in range(_DPW) for c in range(_NFC)]
    fcopies = [
        pltpu.async_copy(
            feature_t_hbm.at[d0 + dd, pl.ds(c * _FCH, _FCH)],
            fch_v.at[k % 2],
            fsem,
        )
        for k, (dd, c) in enumerate(chunks[:2])
    ]
    @pl.when(sid == 0)
    def _():
        pltpu.sync_copy(label_hbm, lab_sh)

    plsc.subcore_barrier()
    lcopy = pltpu.async_copy(lab_sh, lab_v, lsem)

    rcopy.wait()
    lcopy.wait()

    accs = tuple(jnp.zeros((_LANES,), jnp.float32) for _ in range(_UNROLL))
    for k, (dd, c) in enumerate(chunks):
        if c == 0 and dd > 0:
            pltpu.async_copy(centers_t_hbm.at[d0 + dd], row_v, rsem).wait()
        fcopies[k].wait()

        def body(i, a, k=k, c=c):
            new = []
            for u in range(_UNROLL):
                off = i * (_UNROLL * _LANES) + u * _LANES
                lv = lab_v[pl.ds(c * _FCH + off, _LANES)]
                cv = plsc.load_gather(row_v, [lv])
                fv = fch_v[k % 2, pl.ds(off, _LANES)]
                dv = fv - cv
                new.append(a[u] + dv * dv)
            return tuple(new)

        accs = lax.fori_loop(0, _FCH // (_LANES * _UNROLL), body, accs)
        if k + 2 < len(chunks):
            dn, cn = chunks[k + 2]
            fcopies.append(
                pltpu.async_copy(
                    feature_t_hbm.at[d0 + dn, pl.ds(cn * _FCH, _FCH)],
                    fch_v.at[k % 2],
                    fsem,
                )
            )

    part_v[...] = (accs[0] + accs[1]) + (accs[2] + accs[3])
    pltpu.sync_copy(part_v, out_hbm.at[wid])


def kernel(label, feature, centers):
    lab = label.astype(jnp.int32)
    partials = _center_loss_sc(lab, feature.T, centers.T)
    return jnp.sum(partials) * 0.5
